# baseline (speedup 1.0000x reference)
import functools

import jax
import jax.numpy as jnp
from jax import lax
from jax.experimental import pallas as pl
from jax.experimental.pallas import tpu as pltpu
from jax.experimental.pallas import tpu_sc as plsc

N = 10000
F = 128
E = 320000
G = 256
NDEG = 11

NC = 2
NS = 16
NW = NC * NS
EPW = E // NW
NP = 10240
NPT = NP // NS
DR = NP // F

RT = 1000
NT = N // RT


def _make_segsum(with_deg):
    mesh = plsc.VectorSubcoreMesh(
        core_axis_name="c", subcore_axis_name="s", num_cores=NC, num_subcores=NS
    )
    K = 80 if with_deg else 40
    NCHUNK = EPW // K
    NB = 2 if with_deg else 5
    NITER = NCHUNK // NB
    TAIL = NCHUNK - NITER * NB
    out_type = [jax.ShapeDtypeStruct((NC, NP, F), jnp.float32)]
    scratch = (
        [pltpu.VMEM((K,), jnp.int32) for _ in range(NB)]
        + [pltpu.VMEM((K,), jnp.int32) for _ in range(NB)]
        + [pltpu.VMEM((K, F), jnp.float32) for _ in range(NB)]
        + [pltpu.VMEM_SHARED((NP, F), jnp.float32)]
        + [pltpu.SemaphoreType.DMA for _ in range(NB)]
        + [pltpu.SemaphoreType.DMA]
    )
    if with_deg:
        out_type.append(jax.ShapeDtypeStruct((NC, DR, F), jnp.float32))
        scratch += (
            [pltpu.VMEM((K,), jnp.int32) for _ in range(NB)]
            + [pltpu.VMEM((K,), jnp.int32) for _ in range(NB)]
            + [pltpu.VMEM((K, F), jnp.float32) for _ in range(NB)]
            + [pltpu.VMEM_SHARED((DR, F), jnp.float32)]
            + [pltpu.SemaphoreType.DMA for _ in range(NB)]
            + [pltpu.SemaphoreType.DMA]
        )

    @functools.partial(
        pl.kernel, out_type=out_type, mesh=mesh, scratch_types=scratch
    )
    def segsum(feat, srch, dsth, *args):
        if with_deg:
            (zeros_hbm, eye_hbm, part, deg_part) = args[:4]
            args = args[4:]
        else:
            (zeros_hbm, part) = args[:2]
            args = args[2:]
        srcv = args[:NB]
        dstv = args[NB : 2 * NB]
        rows = args[2 * NB : 3 * NB]
        acc_sh = args[3 * NB]
        gsem = args[3 * NB + 1 : 4 * NB + 1]
        ssem = args[4 * NB + 1]
        if with_deg:
            args = args[4 * NB + 2 :]
            rowv = args[:NB]
            colv = args[NB : 2 * NB]
            ohs = args[2 * NB : 3 * NB]
            deg_sh = args[3 * NB]
            esem = args[3 * NB + 1 : 4 * NB + 1]
            dsem = args[4 * NB + 1]
        cid = lax.axis_index("c")
        sid = lax.axis_index("s")
        tid = cid * NS + sid
        pltpu.sync_copy(
            zeros_hbm.at[pl.ds(sid * NPT, NPT)], acc_sh.at[pl.ds(sid * NPT, NPT)]
        )
        if with_deg:
            @pl.when(sid == 0)
            def _():
                pltpu.sync_copy(zeros_hbm.at[pl.ds(0, DR)], deg_sh)

        plsc.subcore_barrier()

        ebase = tid * EPW

        def drain(b):
            pltpu.make_async_copy(rows[b], acc_sh.at[dstv[b]], ssem).wait()
            if with_deg:
                pltpu.make_async_copy(ohs[b], deg_sh.at[rowv[b]], dsem).wait()

        def issue(base, b, gd, ed):
            off = base + b * K
            pltpu.sync_copy(srch.at[pl.ds(off, K)], srcv[b])
            pltpu.sync_copy(dsth.at[pl.ds(off, K)], dstv[b])
            gd.append(pltpu.async_copy(feat.at[srcv[b]], rows[b], gsem[b]))
            if with_deg:
                for j in range(K // 16):
                    dv = dstv[b][pl.ds(j * 16, 16)]
                    rowv[b][pl.ds(j * 16, 16)] = lax.shift_right_logical(dv, 7)
                    colv[b][pl.ds(j * 16, 16)] = lax.bitwise_and(dv, 127)
                ed.append(
                    pltpu.async_copy(eye_hbm.at[colv[b]], ohs[b], esem[b])
                )

        def scatter(b, gd, ed):
            gd[b].wait()
            pltpu.async_copy(rows[b], acc_sh.at[dstv[b]], ssem, add=True)
            if with_deg:
                ed[b].wait()
                pltpu.async_copy(ohs[b], deg_sh.at[rowv[b]], dsem, add=True)

        def body(it, carry):
            base = ebase + it * (NB * K)
            gd, ed = [], []
            for b in range(NB):
                @pl.when(it > 0)
                def _(b=b):
                    drain(b)

                issue(base, b, gd, ed)
            for b in range(NB):
                scatter(b, gd, ed)
            return carry

        lax.fori_loop(0, NITER, body, 0)
        for b in range(NB):
            drain(b)
        if TAIL:
            gd, ed = [], []
            for b in range(TAIL):
                issue(ebase + NITER * NB * K, b, gd, ed)
            for b in range(TAIL):
                scatter(b, gd, ed)
            for b in range(TAIL):
                drain(b)

        plsc.subcore_barrier()
        pltpu.sync_copy(
            acc_sh.at[pl.ds(sid * NPT, NPT)],
            part.at[cid, pl.ds(sid * NPT, NPT)],
        )
        if with_deg:
            @pl.when(sid == 0)
            def _():
                pltpu.sync_copy(deg_sh, deg_part.at[cid])

    return segsum


_segsum_deg = _make_segsum(True)
_segsum = _make_segsum(False)


def _combine1_body(part_ref, x_ref, dp_ref, Wl_ref, bl_ref, Wr_ref,
                   out_ref, deg_ref):
    h = part_ref[0] + part_ref[1]
    xt = x_ref[...]
    deg = jnp.minimum((dp_ref[0] + dp_ref[1]).astype(jnp.int32), NDEG - 1)
    deg_ref[...] = deg
    out_ref[...] = jnp.zeros_like(out_ref)
    for d in range(NDEG):
        m = deg == d

        @pl.when(jnp.any(m))
        def _(d=d, m=m):
            r = (
                jnp.dot(h, Wl_ref[d], preferred_element_type=jnp.float32)
                + jnp.dot(xt, Wr_ref[d], preferred_element_type=jnp.float32)
                + bl_ref[d][None, :]
            )
            out_ref[...] += jnp.where(m, r, 0.0)


def _combine_body(part_ref, x_ref, deg_ref, Wl_ref, bl_ref, Wr_ref, out_ref):
    h = part_ref[0] + part_ref[1]
    xt = x_ref[...]
    deg = deg_ref[...]
    out_ref[...] = jnp.zeros_like(out_ref)
    for d in range(NDEG):
        m = deg == d

        @pl.when(jnp.any(m))
        def _(d=d, m=m):
            r = (
                jnp.dot(h, Wl_ref[d], preferred_element_type=jnp.float32)
                + jnp.dot(xt, Wr_ref[d], preferred_element_type=jnp.float32)
                + bl_ref[d][None, :]
            )
            out_ref[...] += jnp.where(m, r, 0.0)


def _full(shape):
    return pl.BlockSpec(shape, lambda i: tuple(0 for _ in shape))


def _rows(shape):
    return pl.BlockSpec(shape, lambda i: (i,) + tuple(0 for _ in shape[1:]))


def _combine1(part, x, degp, Wl, bl, Wr):
    return pl.pallas_call(
        _combine1_body,
        grid=(NT,),
        in_specs=[
            pl.BlockSpec((NC, RT, F), lambda i: (0, i, 0)),
            _rows((RT, F)),
            pl.BlockSpec((NC, RT, 1), lambda i: (0, i, 0)),
            _full((NDEG, F, F)),
            _full((NDEG, F)),
            _full((NDEG, F, F)),
        ],
        out_specs=[_rows((RT, F)), _rows((RT, 1))],
        out_shape=[
            jax.ShapeDtypeStruct((N, F), jnp.float32),
            jax.ShapeDtypeStruct((N, 1), jnp.int32),
        ],
    )(part, x, degp, Wl, bl, Wr)


def _combine(part, x, deg, Wl, bl, Wr):
    return pl.pallas_call(
        _combine_body,
        grid=(NT,),
        in_specs=[
            pl.BlockSpec((NC, RT, F), lambda i: (0, i, 0)),
            _rows((RT, F)),
            _rows((RT, 1)),
            _full((NDEG, F, F)),
            _full((NDEG, F)),
            _full((NDEG, F, F)),
        ],
        out_specs=_rows((RT, F)),
        out_shape=jax.ShapeDtypeStruct((N, F), jnp.float32),
    )(part, x, deg, Wl, bl, Wr)


def _combine3_body(part_ref, x_ref, deg_ref, batch_ref, Wl_ref, bl_ref,
                   Wr_ref, out_ref, pool_ref):
    _combine_body(part_ref, x_ref, deg_ref, Wl_ref, bl_ref, Wr_ref, out_ref)

    @pl.when(pl.program_id(0) == 0)
    def _():
        pool_ref[...] = jnp.zeros_like(pool_ref)

    oh = (
        batch_ref[...] == lax.broadcasted_iota(jnp.int32, (RT, G), 1)
    ).astype(jnp.float32)
    pool_ref[...] += lax.dot_general(
        oh, out_ref[...], (((0,), (0,)), ((), ())),
        preferred_element_type=jnp.float32,
    )


def _combine3_pool(part, x, deg, batch2, Wl, bl, Wr):
    return pl.pallas_call(
        _combine3_body,
        grid=(NT,),
        in_specs=[
            pl.BlockSpec((NC, RT, F), lambda i: (0, i, 0)),
            _rows((RT, F)),
            _rows((RT, 1)),
            _rows((RT, 1)),
            _full((NDEG, F, F)),
            _full((NDEG, F)),
            _full((NDEG, F, F)),
        ],
        out_specs=[_rows((RT, F)), _full((G, F))],
        out_shape=[
            jax.ShapeDtypeStruct((N, F), jnp.float32),
            jax.ShapeDtypeStruct((G, F), jnp.float32),
        ],
    )(part, x, deg, batch2, Wl, bl, Wr)


def _mlp_body(
    batch_ref, h1_ref, h2_ref, h3_ref, pool_ref,
    L1_ref, b1_ref, L2_ref, b2_ref, L3_ref, b3_ref, out_ref,
):
    bf = jnp.bfloat16
    oh = (
        batch_ref[...] == lax.broadcasted_iota(jnp.int32, (RT, G), 1)
    ).astype(bf)
    hp = jnp.dot(oh, pool_ref[...].astype(bf), preferred_element_type=jnp.float32)
    a = (
        jnp.dot(h1_ref[...].astype(bf), L1_ref[0:F].astype(bf), preferred_element_type=jnp.float32)
        + jnp.dot(h2_ref[...].astype(bf), L1_ref[F : 2 * F].astype(bf), preferred_element_type=jnp.float32)
        + jnp.dot(h3_ref[...].astype(bf), L1_ref[2 * F : 3 * F].astype(bf), preferred_element_type=jnp.float32)
        + jnp.dot(hp.astype(bf), L1_ref[3 * F : 4 * F].astype(bf), preferred_element_type=jnp.float32)
        + b1_ref[...]
    )
    a = _leaky(a).astype(bf)
    b = _leaky(
        jnp.dot(a, L2_ref[...].astype(bf), preferred_element_type=jnp.float32)
        + b2_ref[...]
    ).astype(bf)
    o = jnp.dot(b, L3_ref[...].astype(bf), preferred_element_type=jnp.float32) + b3_ref[...]
    out_ref[...] = jax.nn.sigmoid(o)


def _mlp(batch2, h1, h2, h3, pool, L1, b1, L2, b2, L3, b3):
    H3, H2_ = 3 * F, 2 * F
    return pl.pallas_call(
        _mlp_body,
        grid=(NT,),
        in_specs=[
            _rows((RT, 1)),
            _rows((RT, F)),
            _rows((RT, F)),
            _rows((RT, F)),
            _full((G, F)),
            _full((4 * F, H3)),
            _full((1, H3)),
            _full((H3, H2_)),
            _full((1, H2_)),
            _full((H2_, 1)),
            _full((1, 1)),
        ],
        out_specs=_rows((RT, 1)),
        out_shape=jax.ShapeDtypeStruct((N, 1), jnp.float32),
    )(batch2, h1, h2, h3, pool, L1, b1, L2, b2, L3, b3)


def _leaky(v):
    return jnp.where(v > 0, v, 0.01 * v)


def kernel(x, edge_index, batch, Wl1, bl1, Wr1, Wl2, bl2, Wr2, Wl3, bl3, Wr3,
           L1, b1, L2, b2, L3, b3):
    src = edge_index[0]
    dst = edge_index[1]
    batch2 = batch.reshape(N, 1)

    zeros_f = jnp.zeros((NP, F), jnp.float32)
    eye = jnp.eye(F, dtype=jnp.float32)

    part1, deg_part = _segsum_deg(x, src, dst, zeros_f, eye)
    degp = deg_part.reshape(NC, NP, 1)
    h1, deg = _combine1(part1, x, degp, Wl1, bl1, Wr1)

    (part2,) = _segsum(h1, src, dst, zeros_f)
    h2 = _combine(part2, h1, deg, Wl2, bl2, Wr2)

    (part3,) = _segsum(h2, src, dst, zeros_f)
    h3, pool = _combine3_pool(part3, h2, deg, batch2, Wl3, bl3, Wr3)

    return _mlp(
        batch2, h1, h2, h3, pool,
        L1, b1.reshape(1, -1), L2, b2.reshape(1, -1), L3, b3.reshape(1, 1),
    )


# speedup vs baseline: 1.2486x; 1.2486x over previous
import functools

import jax
import jax.numpy as jnp
from jax import lax
from jax.experimental import pallas as pl
from jax.experimental.pallas import tpu as pltpu
from jax.experimental.pallas import tpu_sc as plsc

N = 10000
F = 128
E = 320000
G = 256
NDEG = 11

NC = 2
NS = 16
NW = NC * NS
EPW = E // NW
NP = 10240
NPT = NP // NS
DR = NP // F

RT = 1000
NT = N // RT


def _make_segsum(with_deg):
    mesh = plsc.VectorSubcoreMesh(
        core_axis_name="c", subcore_axis_name="s", num_cores=NC, num_subcores=NS
    )
    K = 80
    NCHUNK = EPW // K
    NB = 2 if with_deg else 4
    NITER = NCHUNK // NB
    TAIL = NCHUNK - NITER * NB
    out_type = [jax.ShapeDtypeStruct((NC, NP, F), jnp.float32)]
    scratch = (
        [pltpu.VMEM((K,), jnp.int32) for _ in range(NB)]
        + [pltpu.VMEM((K,), jnp.int32) for _ in range(NB)]
        + [pltpu.VMEM((K, F), jnp.float32) for _ in range(NB)]
        + [pltpu.VMEM_SHARED((NP, F), jnp.float32)]
        + [pltpu.SemaphoreType.DMA for _ in range(NB)]
        + [pltpu.SemaphoreType.DMA]
    )
    if with_deg:
        out_type.append(jax.ShapeDtypeStruct((NC, DR, F), jnp.float32))
        scratch += (
            [pltpu.VMEM((K,), jnp.int32) for _ in range(NB)]
            + [pltpu.VMEM((K,), jnp.int32) for _ in range(NB)]
            + [pltpu.VMEM((K, F), jnp.float32) for _ in range(NB)]
            + [pltpu.VMEM_SHARED((DR, F), jnp.float32)]
            + [pltpu.SemaphoreType.DMA for _ in range(NB)]
            + [pltpu.SemaphoreType.DMA]
        )

    @functools.partial(
        pl.kernel, out_type=out_type, mesh=mesh, scratch_types=scratch
    )
    def segsum(feat, srch, dsth, *args):
        if with_deg:
            (zeros_hbm, eye_hbm, part, deg_part) = args[:4]
            args = args[4:]
        else:
            (zeros_hbm, part) = args[:2]
            args = args[2:]
        srcv = args[:NB]
        dstv = args[NB : 2 * NB]
        rows = args[2 * NB : 3 * NB]
        acc_sh = args[3 * NB]
        gsem = args[3 * NB + 1 : 4 * NB + 1]
        ssem = args[4 * NB + 1]
        if with_deg:
            args = args[4 * NB + 2 :]
            rowv = args[:NB]
            colv = args[NB : 2 * NB]
            ohs = args[2 * NB : 3 * NB]
            deg_sh = args[3 * NB]
            esem = args[3 * NB + 1 : 4 * NB + 1]
            dsem = args[4 * NB + 1]
        cid = lax.axis_index("c")
        sid = lax.axis_index("s")
        tid = cid * NS + sid
        pltpu.sync_copy(
            zeros_hbm.at[pl.ds(sid * NPT, NPT)], acc_sh.at[pl.ds(sid * NPT, NPT)]
        )
        if with_deg:
            @pl.when(sid == 0)
            def _():
                pltpu.sync_copy(zeros_hbm.at[pl.ds(0, DR)], deg_sh)

        plsc.subcore_barrier()

        ebase = tid * EPW

        def drain(b):
            pltpu.make_async_copy(rows[b], acc_sh.at[dstv[b]], ssem).wait()
            if with_deg:
                pltpu.make_async_copy(ohs[b], deg_sh.at[rowv[b]], dsem).wait()

        def issue(base, b, gd, ed):
            off = base + b * K
            pltpu.sync_copy(srch.at[pl.ds(off, K)], srcv[b])
            pltpu.sync_copy(dsth.at[pl.ds(off, K)], dstv[b])
            gd.append(pltpu.async_copy(feat.at[srcv[b]], rows[b], gsem[b]))
            if with_deg:
                for j in range(K // 16):
                    dv = dstv[b][pl.ds(j * 16, 16)]
                    rowv[b][pl.ds(j * 16, 16)] = lax.shift_right_logical(dv, 7)
                    colv[b][pl.ds(j * 16, 16)] = lax.bitwise_and(dv, 127)
                ed.append(
                    pltpu.async_copy(eye_hbm.at[colv[b]], ohs[b], esem[b])
                )

        def scatter(b, gd, ed):
            gd[b].wait()
            pltpu.async_copy(rows[b], acc_sh.at[dstv[b]], ssem, add=True)
            if with_deg:
                ed[b].wait()
                pltpu.async_copy(ohs[b], deg_sh.at[rowv[b]], dsem, add=True)

        def body(it, carry):
            base = ebase + it * (NB * K)
            gd, ed = [], []
            for b in range(NB):
                @pl.when(it > 0)
                def _(b=b):
                    drain(b)

                issue(base, b, gd, ed)
            for b in range(NB):
                scatter(b, gd, ed)
            return carry

        lax.fori_loop(0, NITER, body, 0)
        for b in range(NB):
            drain(b)
        if TAIL:
            gd, ed = [], []
            for b in range(TAIL):
                issue(ebase + NITER * NB * K, b, gd, ed)
            for b in range(TAIL):
                scatter(b, gd, ed)
            for b in range(TAIL):
                drain(b)

        plsc.subcore_barrier()
        pltpu.sync_copy(
            acc_sh.at[pl.ds(sid * NPT, NPT)],
            part.at[cid, pl.ds(sid * NPT, NPT)],
        )
        if with_deg:
            @pl.when(sid == 0)
            def _():
                pltpu.sync_copy(deg_sh, deg_part.at[cid])

    return segsum


_segsum_deg = _make_segsum(True)
_segsum = _make_segsum(False)


def _combine1_body(part_ref, x_ref, dp_ref, Wl_ref, bl_ref, Wr_ref,
                   out_ref, deg_ref):
    h = part_ref[0] + part_ref[1]
    xt = x_ref[...]
    deg = jnp.minimum((dp_ref[0] + dp_ref[1]).astype(jnp.int32), NDEG - 1)
    deg_ref[...] = deg
    _degree_banks(h, xt, deg, Wl_ref, bl_ref, Wr_ref, out_ref)


def _degree_banks(h, xt, deg, Wl_ref, bl_ref, Wr_ref, out_ref):
    bf = jnp.bfloat16
    hb = h.astype(bf)
    xb = xt.astype(bf)
    out_ref[...] = jnp.zeros_like(out_ref)
    for d in range(NDEG):
        m = deg == d

        @pl.when(jnp.any(m))
        def _(d=d, m=m):
            r = (
                jnp.dot(hb, Wl_ref[d].astype(bf), preferred_element_type=jnp.float32)
                + jnp.dot(xb, Wr_ref[d].astype(bf), preferred_element_type=jnp.float32)
                + bl_ref[d][None, :]
            )
            out_ref[...] += jnp.where(m, r, 0.0)


def _combine_body(part_ref, x_ref, deg_ref, Wl_ref, bl_ref, Wr_ref, out_ref):
    h = part_ref[0] + part_ref[1]
    xt = x_ref[...]
    deg = deg_ref[...]
    _degree_banks(h, xt, deg, Wl_ref, bl_ref, Wr_ref, out_ref)


def _full(shape):
    return pl.BlockSpec(shape, lambda i: tuple(0 for _ in shape))


def _rows(shape):
    return pl.BlockSpec(shape, lambda i: (i,) + tuple(0 for _ in shape[1:]))


def _combine1(part, x, degp, Wl, bl, Wr):
    return pl.pallas_call(
        _combine1_body,
        grid=(NT,),
        in_specs=[
            pl.BlockSpec((NC, RT, F), lambda i: (0, i, 0)),
            _rows((RT, F)),
            pl.BlockSpec((NC, RT, 1), lambda i: (0, i, 0)),
            _full((NDEG, F, F)),
            _full((NDEG, F)),
            _full((NDEG, F, F)),
        ],
        out_specs=[_rows((RT, F)), _rows((RT, 1))],
        out_shape=[
            jax.ShapeDtypeStruct((N, F), jnp.float32),
            jax.ShapeDtypeStruct((N, 1), jnp.int32),
        ],
    )(part, x, degp, Wl, bl, Wr)


def _combine(part, x, deg, Wl, bl, Wr):
    return pl.pallas_call(
        _combine_body,
        grid=(NT,),
        in_specs=[
            pl.BlockSpec((NC, RT, F), lambda i: (0, i, 0)),
            _rows((RT, F)),
            _rows((RT, 1)),
            _full((NDEG, F, F)),
            _full((NDEG, F)),
            _full((NDEG, F, F)),
        ],
        out_specs=_rows((RT, F)),
        out_shape=jax.ShapeDtypeStruct((N, F), jnp.float32),
    )(part, x, deg, Wl, bl, Wr)


def _combine3_body(part_ref, x_ref, deg_ref, batch_ref, Wl_ref, bl_ref,
                   Wr_ref, out_ref, pool_ref):
    _combine_body(part_ref, x_ref, deg_ref, Wl_ref, bl_ref, Wr_ref, out_ref)

    @pl.when(pl.program_id(0) == 0)
    def _():
        pool_ref[...] = jnp.zeros_like(pool_ref)

    oh = (
        batch_ref[...] == lax.broadcasted_iota(jnp.int32, (RT, G), 1)
    ).astype(jnp.float32)
    pool_ref[...] += lax.dot_general(
        oh, out_ref[...], (((0,), (0,)), ((), ())),
        preferred_element_type=jnp.float32,
    )


def _combine3_pool(part, x, deg, batch2, Wl, bl, Wr):
    return pl.pallas_call(
        _combine3_body,
        grid=(NT,),
        in_specs=[
            pl.BlockSpec((NC, RT, F), lambda i: (0, i, 0)),
            _rows((RT, F)),
            _rows((RT, 1)),
            _rows((RT, 1)),
            _full((NDEG, F, F)),
            _full((NDEG, F)),
            _full((NDEG, F, F)),
        ],
        out_specs=[_rows((RT, F)), _full((G, F))],
        out_shape=[
            jax.ShapeDtypeStruct((N, F), jnp.float32),
            jax.ShapeDtypeStruct((G, F), jnp.float32),
        ],
    )(part, x, deg, batch2, Wl, bl, Wr)


def _mlp_body(
    batch_ref, h1_ref, h2_ref, h3_ref, pool_ref,
    L1_ref, b1_ref, L2_ref, b2_ref, L3_ref, b3_ref, out_ref,
):
    bf = jnp.bfloat16
    oh = (
        batch_ref[...] == lax.broadcasted_iota(jnp.int32, (RT, G), 1)
    ).astype(bf)
    hp = jnp.dot(oh, pool_ref[...].astype(bf), preferred_element_type=jnp.float32)
    a = (
        jnp.dot(h1_ref[...].astype(bf), L1_ref[0:F].astype(bf), preferred_element_type=jnp.float32)
        + jnp.dot(h2_ref[...].astype(bf), L1_ref[F : 2 * F].astype(bf), preferred_element_type=jnp.float32)
        + jnp.dot(h3_ref[...].astype(bf), L1_ref[2 * F : 3 * F].astype(bf), preferred_element_type=jnp.float32)
        + jnp.dot(hp.astype(bf), L1_ref[3 * F : 4 * F].astype(bf), preferred_element_type=jnp.float32)
        + b1_ref[...]
    )
    a = _leaky(a).astype(bf)
    b = _leaky(
        jnp.dot(a, L2_ref[...].astype(bf), preferred_element_type=jnp.float32)
        + b2_ref[...]
    ).astype(bf)
    o = jnp.dot(b, L3_ref[...].astype(bf), preferred_element_type=jnp.float32) + b3_ref[...]
    out_ref[...] = jax.nn.sigmoid(o)


def _mlp(batch2, h1, h2, h3, pool, L1, b1, L2, b2, L3, b3):
    H3, H2_ = 3 * F, 2 * F
    return pl.pallas_call(
        _mlp_body,
        grid=(NT,),
        in_specs=[
            _rows((RT, 1)),
            _rows((RT, F)),
            _rows((RT, F)),
            _rows((RT, F)),
            _full((G, F)),
            _full((4 * F, H3)),
            _full((1, H3)),
            _full((H3, H2_)),
            _full((1, H2_)),
            _full((H2_, 1)),
            _full((1, 1)),
        ],
        out_specs=_rows((RT, 1)),
        out_shape=jax.ShapeDtypeStruct((N, 1), jnp.float32),
    )(batch2, h1, h2, h3, pool, L1, b1, L2, b2, L3, b3)


def _leaky(v):
    return jnp.where(v > 0, v, 0.01 * v)


def kernel(x, edge_index, batch, Wl1, bl1, Wr1, Wl2, bl2, Wr2, Wl3, bl3, Wr3,
           L1, b1, L2, b2, L3, b3):
    src = edge_index[0]
    dst = edge_index[1]
    batch2 = batch.reshape(N, 1)

    zeros_f = jnp.zeros((NP, F), jnp.float32)
    eye = jnp.eye(F, dtype=jnp.float32)

    part1, deg_part = _segsum_deg(x, src, dst, zeros_f, eye)
    degp = deg_part.reshape(NC, NP, 1)
    h1, deg = _combine1(part1, x, degp, Wl1, bl1, Wr1)

    (part2,) = _segsum(h1, src, dst, zeros_f)
    h2 = _combine(part2, h1, deg, Wl2, bl2, Wr2)

    (part3,) = _segsum(h2, src, dst, zeros_f)
    h3, pool = _combine3_pool(part3, h2, deg, batch2, Wl3, bl3, Wr3)

    return _mlp(
        batch2, h1, h2, h3, pool,
        L1, b1.reshape(1, -1), L2, b2.reshape(1, -1), L3, b3.reshape(1, 1),
    )
